# Initial kernel scaffold; baseline (speedup 1.0000x reference)
import jax
import jax.numpy as jnp
from jax.experimental import pallas as pl


def kernel(category_ids, embedding_weight):
    raise NotImplementedError("write your pallas kernel here")


# speedup vs baseline: 1.0953x; 1.0953x over previous
import functools

import jax
import jax.numpy as jnp
from jax import lax
from jax.experimental import pallas as pl
from jax.experimental.pallas import tpu as pltpu
from jax.experimental.pallas import tpu_sc as plsc

L = 16


@functools.lru_cache(maxsize=None)
def _make_encoder(B, C, V, D):
    info = plsc.get_sparse_core_info()
    NC, NS = info.num_cores, info.num_subcores
    NW = NC * NS
    b_per_w = B // NW
    BC = 32
    NCH = b_per_w // BC
    ROWS = BC * C
    G = 104
    NG = ROWS // G
    KD = D // L

    mesh = plsc.VectorSubcoreMesh(core_axis_name="c", subcore_axis_name="s")

    @functools.partial(
        pl.kernel,
        mesh=mesh,
        compiler_params=pltpu.CompilerParams(use_tc_tiling_on_sc=False, needs_layout_passes=False),
        out_type=jax.ShapeDtypeStruct((B, D), jnp.float32),
        scratch_types=[
            pltpu.VMEM((ROWS,), jnp.int32),
            pltpu.VMEM((ROWS, D), jnp.float32),
            pltpu.VMEM((1, D), jnp.float32),
            pltpu.VMEM((BC, D), jnp.float32),
            pltpu.SemaphoreType.DMA,
        ],
    )
    def enc(ids_hbm, tab_hbm, out_hbm, idx_v, buf_v, r0_v, out_v, sem):
        wid = lax.axis_index("s") * NC + lax.axis_index("c")
        row_base = wid * b_per_w
        pltpu.sync_copy(tab_hbm.at[pl.ds(0, 1)], r0_v)
        r0 = [r0_v[0, k * L:(k + 1) * L] for k in range(KD)]
        lanes = lax.iota(jnp.int32, L)
        ovw = jnp.minimum(jnp.maximum(lanes - (2 * L - C - 1), 0), 1)

        def chunk_body(ch, carry):
            b0 = row_base + ch * BC
            pltpu.sync_copy(ids_hbm.at[pl.ds(b0 * C, ROWS)], idx_v)
            cps = [
                pltpu.async_copy(
                    tab_hbm.at[idx_v.at[pl.ds(g * G, G)]],
                    buf_v.at[pl.ds(g * G, G)], sem)
                for g in range(NG)
            ]
            for cp in cps:
                cp.wait()

            def b_body(b, carry2):
                r = b * C
                accs = [buf_v[r, k * L:(k + 1) * L] for k in range(KD)]
                for c in range(1, C):
                    for k in range(KD):
                        accs[k] = accs[k] + buf_v[r + c, k * L:(k + 1) * L]
                v0 = idx_v[pl.ds(r, L)]
                v1 = idx_v[pl.ds(r + C - L, L)]
                z0 = 1 - jnp.minimum(v0, 1)
                z1 = (1 - jnp.minimum(v1, 1)) * ovw
                n0 = jnp.sum(z0 + z1)
                n0v = jnp.full((L,), n0).astype(jnp.float32)
                inv = 1.0 / ((float(C) - n0v) + 1e-8)
                for k in range(KD):
                    out_v[b, k * L:(k + 1) * L] = (accs[k] - n0v * r0[k]) * inv
                return carry2

            lax.fori_loop(0, BC, b_body, 0)
            pltpu.sync_copy(out_v, out_hbm.at[pl.ds(b0, BC)])
            return carry

        lax.fori_loop(0, NCH, chunk_body, 0)

    return enc


def kernel(category_ids, embedding_weight):
    B, C = category_ids.shape
    V, D = embedding_weight.shape
    ids_flat = category_ids.reshape(-1).astype(jnp.int32)
    return _make_encoder(B, C, V, D)(ids_flat, embedding_weight)
